# Initial kernel scaffold; baseline (speedup 1.0000x reference)
import jax
import jax.numpy as jnp
from jax.experimental import pallas as pl


def kernel(input_ids, embedding_weights):
    raise NotImplementedError("write your pallas kernel here")


# speedup vs baseline: 2.3076x; 2.3076x over previous
import functools

import jax
import jax.numpy as jnp
from jax import lax
from jax.experimental import pallas as pl
from jax.experimental.pallas import tpu as pltpu
from jax.experimental.pallas import tpu_sc as plsc

PAD = 1
BSZ = 4
SEQ = 8192
D = 768
L = 16
NW = 32
CHUNK = (BSZ * SEQ) // NW
WPR = SEQ // CHUNK
BLK = 64
NBLK = CHUNK // BLK
VSTEPS = CHUNK // L


def _sc_kernel(ids_hbm, table_hbm, out_hbm, ids_row, idx_v, buf0, buf1,
               gsem0, gsem1, ssem0, ssem1):
    wid = lax.axis_index("s") * 2 + lax.axis_index("c")
    row = wid // WPR
    chunk = wid % WPR

    pltpu.sync_copy(ids_hbm.at[row], ids_row)

    def acc_body(j, carry):
        v = ids_row[pl.ds(j * L, L)]
        return carry + jnp.sum(jnp.minimum(jnp.abs(v - PAD), 1))

    carry = lax.fori_loop(0, chunk * VSTEPS, acc_body,
                          jnp.zeros((), jnp.int32))

    base_step = chunk * VSTEPS

    def emit_body(j2, carry):
        v = ids_row[pl.ds((base_step + j2) * L, L)]
        mi = jnp.minimum(jnp.abs(v - PAD), 1)
        cs = plsc.cumsum(mi)
        idx_v[pl.ds(j2 * L, L)] = (cs + carry) * mi + PAD
        return carry + jnp.sum(mi)

    lax.fori_loop(0, VSTEPS, emit_body, carry)

    out_base = wid * CHUNK
    bufs = (buf0, buf1)
    gsems = (gsem0, gsem1)
    ssems = (ssem0, ssem1)

    def gather(t):
        return pltpu.async_copy(
            table_hbm.at[idx_v.at[pl.ds(t * BLK, BLK)]], bufs[t % 2],
            gsems[t % 2])

    def scatter(t):
        return pltpu.async_copy(
            bufs[t % 2], out_hbm.at[pl.ds(out_base + t * BLK, BLK)],
            ssems[t % 2])

    g = {0: gather(0)}
    s = {}
    for t in range(NBLK):
        g[t].wait()
        s[t] = scatter(t)
        if t + 1 < NBLK:
            if t >= 1:
                s[t - 1].wait()
            g[t + 1] = gather(t + 1)
    s[NBLK - 2].wait()
    s[NBLK - 1].wait()


@jax.jit
def kernel(input_ids, embedding_weights):
    mesh = plsc.VectorSubcoreMesh(core_axis_name="c", subcore_axis_name="s")
    run = functools.partial(
        pl.kernel,
        mesh=mesh,
        compiler_params=pltpu.CompilerParams(needs_layout_passes=False),
        out_type=jax.ShapeDtypeStruct((BSZ * SEQ, D), jnp.float32),
        scratch_types=[
            pltpu.VMEM((SEQ,), jnp.int32),
            pltpu.VMEM((CHUNK,), jnp.int32),
            pltpu.VMEM((BLK, D), jnp.float32),
            pltpu.VMEM((BLK, D), jnp.float32),
            pltpu.SemaphoreType.DMA,
            pltpu.SemaphoreType.DMA,
            pltpu.SemaphoreType.DMA,
            pltpu.SemaphoreType.DMA,
        ],
    )(_sc_kernel)
    out = run(input_ids.astype(jnp.int32), embedding_weights)
    return out.reshape(BSZ, SEQ, D)


# speedup vs baseline: 2.3154x; 1.0034x over previous
import functools

import jax
import jax.numpy as jnp
from jax import lax
from jax.experimental import pallas as pl
from jax.experimental.pallas import tpu as pltpu
from jax.experimental.pallas import tpu_sc as plsc

PAD = 1
BSZ = 4
SEQ = 8192
D = 768
L = 16
NW = 32
CHUNK = (BSZ * SEQ) // NW
WPR = SEQ // CHUNK
BLK = 64
NBLK = CHUNK // BLK
VSTEPS = CHUNK // L


def _sc_kernel(ids_hbm, table_hbm, out_hbm, ids_row, idx_v, buf0, buf1,
               gsem0, gsem1, ssem0, ssem1):
    wid = lax.axis_index("s") * 2 + lax.axis_index("c")
    row = wid // WPR
    chunk = wid % WPR

    pltpu.sync_copy(ids_hbm.at[row], ids_row)

    def acc_body(j, carry_v):
        v = ids_row[pl.ds(j * L, L)]
        return carry_v + jnp.minimum(jnp.abs(v - PAD), 1)

    carry_v = lax.fori_loop(0, chunk * VSTEPS, acc_body,
                            jnp.zeros((L,), jnp.int32))
    carry = jnp.sum(carry_v)

    base_step = chunk * VSTEPS

    def emit_body(j2, carry):
        v = ids_row[pl.ds((base_step + j2) * L, L)]
        mi = jnp.minimum(jnp.abs(v - PAD), 1)
        cs = plsc.cumsum(mi)
        idx_v[pl.ds(j2 * L, L)] = (cs + carry) * mi + PAD
        return carry + jnp.sum(mi)

    lax.fori_loop(0, VSTEPS, emit_body, carry)

    out_base = wid * CHUNK
    bufs = (buf0, buf1)
    gsems = (gsem0, gsem1)
    ssems = (ssem0, ssem1)

    def gather(t):
        return pltpu.async_copy(
            table_hbm.at[idx_v.at[pl.ds(t * BLK, BLK)]], bufs[t % 2],
            gsems[t % 2])

    def scatter(t):
        return pltpu.async_copy(
            bufs[t % 2], out_hbm.at[pl.ds(out_base + t * BLK, BLK)],
            ssems[t % 2])

    g = {0: gather(0)}
    s = {}
    for t in range(NBLK):
        g[t].wait()
        s[t] = scatter(t)
        if t + 1 < NBLK:
            if t >= 1:
                s[t - 1].wait()
            g[t + 1] = gather(t + 1)
    s[NBLK - 2].wait()
    s[NBLK - 1].wait()


@jax.jit
def kernel(input_ids, embedding_weights):
    mesh = plsc.VectorSubcoreMesh(core_axis_name="c", subcore_axis_name="s")
    run = functools.partial(
        pl.kernel,
        mesh=mesh,
        compiler_params=pltpu.CompilerParams(needs_layout_passes=False),
        out_type=jax.ShapeDtypeStruct((BSZ * SEQ, D), jnp.float32),
        scratch_types=[
            pltpu.VMEM((SEQ,), jnp.int32),
            pltpu.VMEM((CHUNK,), jnp.int32),
            pltpu.VMEM((BLK, D), jnp.float32),
            pltpu.VMEM((BLK, D), jnp.float32),
            pltpu.SemaphoreType.DMA,
            pltpu.SemaphoreType.DMA,
            pltpu.SemaphoreType.DMA,
            pltpu.SemaphoreType.DMA,
        ],
    )(_sc_kernel)
    out = run(input_ids.astype(jnp.int32), embedding_weights)
    return out.reshape(BSZ, SEQ, D)


# speedup vs baseline: 2.3585x; 1.0186x over previous
import functools

import jax
import jax.numpy as jnp
from jax import lax
from jax.experimental import pallas as pl
from jax.experimental.pallas import tpu as pltpu
from jax.experimental.pallas import tpu_sc as plsc

PAD = 1
BSZ = 4
SEQ = 8192
D = 768
L = 16
NW = 32
CHUNK = (BSZ * SEQ) // NW
WPR = SEQ // CHUNK
BLK = 32
NBLK = CHUNK // BLK
NBUF = 4
VSTEPS = CHUNK // L


def _sc_kernel(ids_hbm, table_hbm, out_hbm, ids_row, idx_v,
               buf0, buf1, buf2, buf3,
               gsem0, gsem1, gsem2, gsem3, ssem0, ssem1, ssem2, ssem3):
    wid = lax.axis_index("s") * 2 + lax.axis_index("c")
    row = wid // WPR
    chunk = wid % WPR

    pltpu.sync_copy(ids_hbm.at[row], ids_row)

    def acc_body(j, carry_v):
        v = ids_row[pl.ds(j * L, L)]
        return carry_v + jnp.minimum(jnp.abs(v - PAD), 1)

    carry_v = lax.fori_loop(0, chunk * VSTEPS, acc_body,
                            jnp.zeros((L,), jnp.int32))
    carry = jnp.sum(carry_v)

    base_step = chunk * VSTEPS

    def emit_body(j2, carry):
        v = ids_row[pl.ds((base_step + j2) * L, L)]
        mi = jnp.minimum(jnp.abs(v - PAD), 1)
        cs = plsc.cumsum(mi)
        idx_v[pl.ds(j2 * L, L)] = (cs + carry) * mi + PAD
        return carry + jnp.sum(mi)

    lax.fori_loop(0, VSTEPS, emit_body, carry)

    out_base = wid * CHUNK
    bufs = (buf0, buf1, buf2, buf3)
    gsems = (gsem0, gsem1, gsem2, gsem3)
    ssems = (ssem0, ssem1, ssem2, ssem3)

    def gather(t):
        return pltpu.async_copy(
            table_hbm.at[idx_v.at[pl.ds(t * BLK, BLK)]], bufs[t % NBUF],
            gsems[t % NBUF])

    def scatter(t):
        return pltpu.async_copy(
            bufs[t % NBUF], out_hbm.at[pl.ds(out_base + t * BLK, BLK)],
            ssems[t % NBUF])

    g = {}
    s = {}
    for t in range(NBUF - 1):
        g[t] = gather(t)
    for t in range(NBLK):
        g[t].wait()
        s[t] = scatter(t)
        nt = t + NBUF - 1
        if nt < NBLK:
            if nt >= NBUF:
                s[nt - NBUF].wait()
            g[nt] = gather(nt)
    for t in range(NBLK - NBUF, NBLK):
        s[t].wait()


@jax.jit
def kernel(input_ids, embedding_weights):
    mesh = plsc.VectorSubcoreMesh(core_axis_name="c", subcore_axis_name="s")
    run = functools.partial(
        pl.kernel,
        mesh=mesh,
        compiler_params=pltpu.CompilerParams(needs_layout_passes=False),
        out_type=jax.ShapeDtypeStruct((BSZ * SEQ, D), jnp.float32),
        scratch_types=[
            pltpu.VMEM((SEQ,), jnp.int32),
            pltpu.VMEM((CHUNK,), jnp.int32),
            pltpu.VMEM((BLK, D), jnp.float32),
            pltpu.VMEM((BLK, D), jnp.float32),
            pltpu.VMEM((BLK, D), jnp.float32),
            pltpu.VMEM((BLK, D), jnp.float32),
            pltpu.SemaphoreType.DMA,
            pltpu.SemaphoreType.DMA,
            pltpu.SemaphoreType.DMA,
            pltpu.SemaphoreType.DMA,
            pltpu.SemaphoreType.DMA,
            pltpu.SemaphoreType.DMA,
            pltpu.SemaphoreType.DMA,
            pltpu.SemaphoreType.DMA,
        ],
    )(_sc_kernel)
    out = run(input_ids.astype(jnp.int32), embedding_weights)
    return out.reshape(BSZ, SEQ, D)
